# baseline (speedup 1.0000x reference)
import functools

import jax
import jax.numpy as jnp
from jax import lax
from jax.experimental import pallas as pl
from jax.experimental.pallas import tpu as pltpu
from jax.experimental.pallas import tpu_sc as plsc

NC, NS, L = 2, 16, 16
B = 16
V = 100001
RNN = 2048
HALF = 50000
WIN = 50048
UNROLL = 8
TOTAL = B * V
NEG = -1e10
NINF = -3e38
TOKBITS = 17

NBLK = WIN // (UNROLL * L)
NBLK2 = (NBLK + UNROLL - 1) // UNROLL
CHUNK_BLKS = (98, 98, 98, 97)

_mesh = plsc.VectorSubcoreMesh(core_axis_name="c", subcore_axis_name="s")


def _rev(x):
    return lax.rev(x, (0,))


def _merge_sorted(av, ai, bv, bi):
    rbv, rbi = _rev(bv), _rev(bi)
    take = rbv > av
    nv = jnp.where(take, rbv, av)
    ni = jnp.where(take, rbi, ai)
    return plsc.sort_key_val(nv, ni)


def _insert16(tv, ti, v, vi):
    sv, si = plsc.sort_key_val(v, vi)
    rsv, rsi = _rev(sv), _rev(si)
    take = rsv > tv
    nv = jnp.where(take, rsv, tv)
    ni = jnp.where(take, rsi, ti)
    tv2, ti2 = plsc.sort_key_val(nv, ni)
    return tv2, ti2, tv2[0]


@functools.partial(
    pl.kernel,
    out_type=(
        jax.ShapeDtypeStruct((NS, NC, L), jnp.float32),
        jax.ShapeDtypeStruct((NS, NC, L), jnp.int32),
    ),
    mesh=_mesh,
    scratch_types=[
        pltpu.VMEM((WIN,), jnp.float32),
        pltpu.VMEM((NBLK2 * UNROLL * L,), jnp.float32),
        pltpu.VMEM((NBLK2 * L,), jnp.float32),
        pltpu.VMEM((L,), jnp.int32),
        pltpu.VMEM((L,), jnp.float32),
        pltpu.VMEM((L,), jnp.int32),
        pltpu.SemaphoreType.DMA,
        pltpu.SemaphoreType.DMA,
        pltpu.SemaphoreType.DMA,
        pltpu.SemaphoreType.DMA,
    ],
    compiler_params=pltpu.CompilerParams(needs_layout_passes=False),
)
def _scan(lp_hbm, forbid_hbm, ovals_hbm, oidx_hbm,
          buf, g1, g2, prev_v, vals_v, idx_v, s0, s1, s2, s3):
    c = lax.axis_index("c")
    s = lax.axis_index("s")
    lane = lax.iota(jnp.int32, L)
    negs = jnp.full((L,), NEG, jnp.float32)

    row_base = s * V
    lo = row_base + c * HALF
    astart = jnp.minimum((lo // 8) * 8, TOTAL - WIN)
    d = lo - astart

    sems = (s0, s1, s2, s3)
    copies = []
    base = 0
    for nb in CHUNK_BLKS:
        n = nb * UNROLL * L
        copies.append(pltpu.async_copy(
            lp_hbm.at[pl.ds(astart + base, n)], buf.at[pl.ds(base, n)],
            sems[len(copies)]))
        base += n

    pltpu.sync_copy(forbid_hbm.at[pl.ds(s * L, L)], prev_v)
    prev_flat = prev_v[...]
    pidx = prev_flat - astart

    def pass_a(i, _):
        bb = i * (UNROLL * L)
        g = buf[pl.ds(bb, L)]
        for k in range(1, UNROLL):
            g = jnp.maximum(g, buf[pl.ds(bb + k * L, L)])
        g1[pl.ds(i * L, L)] = g
        return 0

    blk0 = 0
    base = 0
    for ci, nb in enumerate(CHUNK_BLKS):
        n = nb * UNROLL * L
        copies[ci].wait()
        if ci == 0:
            for j in range(3):
                idxh = j * L + lane
                plsc.store_scatter(buf, [idxh], negs, mask=idxh < d)
        if ci == len(CHUNK_BLKS) - 1:
            for j in range(3):
                idxt = (HALF // L + j) * L + lane
                plsc.store_scatter(buf, [idxt], negs,
                                   mask=idxt >= HALF + d)
        pmask = (pidx >= base) & (pidx < base + n)
        plsc.store_scatter(buf, [jnp.clip(pidx, 0, WIN - 1)], negs,
                           mask=pmask)
        lax.fori_loop(blk0, blk0 + nb, pass_a, 0)
        blk0 += nb
        base += n

    g1[pl.ds(NBLK * L, L)] = negs

    def pass_a2(i, _):
        bb = i * (UNROLL * L)
        g = g1[pl.ds(bb, L)]
        for k in range(1, UNROLL):
            g = jnp.maximum(g, g1[pl.ds(bb + k * L, L)])
        g2[pl.ds(i * L, L)] = g
        return 0

    lax.fori_loop(0, NBLK2, pass_a2, 0)

    def pass_b(i, tvb):
        blk = g2[pl.ds(i * L, L)]

        def ins(t):
            return jnp.sort(jnp.maximum(t, _rev(jnp.sort(blk))))

        cnt = plsc.all_reduce_population_count(blk > tvb[0])[0]
        return lax.cond(cnt > 0, ins, lambda t: t, tvb)

    tvb = lax.fori_loop(0, NBLK2, pass_b,
                        jnp.full((L,), NINF, jnp.float32))
    thr = tvb[0]

    def hits(v):
        return plsc.all_reduce_population_count(v >= thr)[0]

    def pass_c(j, carry):
        def descend(c2):
            for a in range(UNROLL):
                gi = j * UNROLL + a

                def down2(c3, gi=gi):
                    for k in range(UNROLL):
                        bb = (gi * UNROLL + k) * L

                        def ins(c4, bb=bb):
                            fvec = (astart + bb) + lane
                            v = buf[pl.ds(bb, L)]
                            return _insert16(c4[0], c4[1], v,
                                             fvec - row_base)[:2]

                        c3 = lax.cond(hits(buf[pl.ds(bb, L)]) > 0,
                                      ins, lambda c4: c4, c3)
                    return c3

                c2 = lax.cond((gi < NBLK) & (hits(g1[pl.ds(gi * L, L)]) > 0),
                              down2, lambda c3: c3, c2)
            return c2

        return lax.cond(hits(g2[pl.ds(j * L, L)]) > 0,
                        descend, lambda c2: c2, carry)

    init = (jnp.full((L,), NINF, jnp.float32), jnp.zeros((L,), jnp.int32))
    tv, ti = lax.fori_loop(0, NBLK2, pass_c, init)

    vals_v[...] = tv
    idx_v[...] = ti
    pltpu.sync_copy(vals_v, ovals_hbm.at[s, c])
    pltpu.sync_copy(idx_v, oidx_hbm.at[s, c])


@functools.partial(
    pl.kernel,
    out_type=(
        jax.ShapeDtypeStruct((B,), jnp.int32),
        jax.ShapeDtypeStruct((B,), jnp.float32),
        jax.ShapeDtypeStruct((B,), jnp.float32),
        jax.ShapeDtypeStruct((2 * B, RNN), jnp.float32),
        jax.ShapeDtypeStruct((2 * B, RNN), jnp.float32),
    ),
    mesh=_mesh,
    scratch_types=[
        pltpu.VMEM((NC, L), jnp.float32),
        pltpu.VMEM((NC, L), jnp.int32),
        pltpu.VMEM((L,), jnp.float32),
        pltpu.VMEM((L,), jnp.float32),
        pltpu.VMEM((B * L,), jnp.float32),
        pltpu.VMEM((L,), jnp.float32),
        pltpu.VMEM((L,), jnp.int32),
        pltpu.VMEM((L,), jnp.int32),
        pltpu.VMEM((L,), jnp.float32),
        pltpu.VMEM((L,), jnp.float32),
        pltpu.VMEM((4,), jnp.int32),
        pltpu.VMEM((4, RNN), jnp.float32),
        pltpu.VMEM((17 * L,), jnp.float32),
        pltpu.VMEM((17 * L,), jnp.int32),
        pltpu.VMEM((8 * NS,), jnp.int32),
        pltpu.VMEM_SHARED((17 * L,), jnp.float32),
        pltpu.VMEM_SHARED((17 * L,), jnp.int32),
        pltpu.VMEM_SHARED((8 * NS,), jnp.int32),
        pltpu.SemaphoreType.DMA,
    ],
    compiler_params=pltpu.CompilerParams(needs_layout_passes=False),
)
def _merge(vals_hbm, idx_hbm, bsum_hbm, bsumb_hbm, eos_hbm, prev_hbm,
           sh_hbm, sc_hbm,
           tok_out, p_out, r_out, nh_out, nc_out,
           hv_v, hi_v, bs_v, bsum_v, bsumb_v, eos_v, prev_v, tok_v, p_v, r_v,
           q4_v, rows_v, mv_v, mi_v, q2_v, shv, shi, qsh, sem):
    c = lax.axis_index("c")
    s = lax.axis_index("s")
    lane = lax.iota(jnp.int32, L)

    @pl.when(c == 0)
    def _core0():
        pltpu.sync_copy(vals_hbm.at[s], hv_v)
        pltpu.sync_copy(idx_hbm.at[s], hi_v)
        pltpu.sync_copy(bsumb_hbm.at[pl.ds(s * L, L)], bs_v)
        mv, mi = _merge_sorted(hv_v[0], hi_v[0], hv_v[1], hi_v[1])
        p_v[...] = mv + bs_v[...]
        tok_v[...] = mi | lax.shift_left(s, TOKBITS)
        pltpu.sync_copy(p_v, shv.at[pl.ds(s * L, L)])
        pltpu.sync_copy(tok_v, shi.at[pl.ds(s * L, L)])

        @pl.when(s == 0)
        def _eos():
            pltpu.sync_copy(eos_hbm, eos_v)
            pltpu.sync_copy(prev_hbm, prev_v)
            pltpu.sync_copy(bsum_hbm, bsum_v)
            pltpu.sync_copy(bsumb_hbm, bsumb_v)
            ev = (jnp.where(prev_v[...] == V - 1, NEG, eos_v[...])
                  - 1000.0 + bsum_v[...])
            epk = jnp.left_shift(lane, TOKBITS) | (V - 1)
            sv, si = plsc.sort_key_val(ev, epk)
            p_v[...] = sv
            tok_v[...] = si
            pltpu.sync_copy(p_v, shv.at[pl.ds(B * L, L)])
            pltpu.sync_copy(tok_v, shi.at[pl.ds(B * L, L)])

        plsc.subcore_barrier()

        @pl.when(s < 4)
        def _fold4():
            pltpu.sync_copy(shv.at[pl.ds(4 * s * L, 4 * L)],
                            mv_v.at[pl.ds(0, 4 * L)])
            pltpu.sync_copy(shi.at[pl.ds(4 * s * L, 4 * L)],
                            mi_v.at[pl.ds(0, 4 * L)])
            av = mv_v[pl.ds(0, L)]
            ai = mi_v[pl.ds(0, L)]
            for k in range(1, 4):
                av, ai = _merge_sorted(av, ai, mv_v[pl.ds(k * L, L)],
                                       mi_v[pl.ds(k * L, L)])
            p_v[...] = av
            tok_v[...] = ai
            pltpu.sync_copy(p_v, shv.at[pl.ds(4 * s * L, L)])
            pltpu.sync_copy(tok_v, shi.at[pl.ds(4 * s * L, L)])

        plsc.subcore_barrier()

        @pl.when(s == 0)
        def _lead():
            pltpu.sync_copy(shv, mv_v)
            pltpu.sync_copy(shi, mi_v)
            m1v, m1i = _merge_sorted(mv_v[pl.ds(0, L)], mi_v[pl.ds(0, L)],
                                     mv_v[pl.ds(4 * L, L)],
                                     mi_v[pl.ds(4 * L, L)])
            m2v, m2i = _merge_sorted(mv_v[pl.ds(8 * L, L)],
                                     mi_v[pl.ds(8 * L, L)],
                                     mv_v[pl.ds(12 * L, L)],
                                     mi_v[pl.ds(12 * L, L)])
            gv, gi = _merge_sorted(m1v, m1i, m2v, m2i)
            gv, gi = _merge_sorted(gv, gi, mv_v[pl.ds(B * L, L)],
                                   mi_v[pl.ds(B * L, L)])
            top_p = _rev(gv)
            pk = _rev(gi)
            token = pk & ((1 << TOKBITS) - 1)
            q = lax.shift_right_logical(pk, TOKBITS)
            bq = jnp.zeros((L,), jnp.float32)
            for b in range(B):
                bq = jnp.where(q == b, bsumb_v[pl.ds(b * L, L)], bq)
            r = top_p - bq

            tok_v[...] = token
            p_v[...] = top_p
            r_v[...] = r
            dest = 8 * (lane // 4) + (lane % 4)
            plsc.store_scatter(q2_v, [dest], q)
            plsc.store_scatter(q2_v, [dest + 32], q + B)
            plsc.store_scatter(q2_v, [dest + 64], q)
            plsc.store_scatter(q2_v, [dest + 96], q + B)
            pltpu.sync_copy(q2_v, qsh)
            pltpu.sync_copy(tok_v, tok_out)
            pltpu.sync_copy(p_v, p_out)
            pltpu.sync_copy(r_v, r_out)

        plsc.subcore_barrier()

        for k in range(NS):
            @pl.when(s == k)
            def _gather(k=k):
                pltpu.sync_copy(qsh.at[pl.ds(8 * k, 4)], q4_v)
                src = sh_hbm if k < 8 else sc_hbm
                dst = nh_out if k < 8 else nc_out
                dbase = 4 * k if k < 8 else 4 * k - 32
                pltpu.async_copy(src.at[q4_v], rows_v, sem).wait()
                pltpu.sync_copy(rows_v, dst.at[pl.ds(dbase, 4)])


def kernel(logprobs, beam_logprobs_sum, state_h, state_c, prev_tokens):
    lp = logprobs.astype(jnp.float32)
    prev = prev_tokens.astype(jnp.int32)
    bsum = beam_logprobs_sum.astype(jnp.float32)
    eos = lp[:, V - 1]
    forbid = jnp.broadcast_to(
        (prev + jnp.arange(B, dtype=jnp.int32) * V)[:, None], (B, L)
    ).reshape(-1)
    bsumb = jnp.broadcast_to(bsum[:, None], (B, L)).reshape(-1)
    vals, idx = _scan(lp.reshape(-1), forbid)
    token, top_p, r, nh, nc = _merge(
        vals, idx, bsum, bsumb, eos, prev,
        state_h.reshape(2 * B, RNN), state_c.reshape(2 * B, RNN))
    return (token, top_p, r,
            nh.reshape(2, B, RNN), nc.reshape(2, B, RNN))


# speedup vs baseline: 1.0301x; 1.0301x over previous
import functools

import jax
import jax.numpy as jnp
from jax import lax
from jax.experimental import pallas as pl
from jax.experimental.pallas import tpu as pltpu
from jax.experimental.pallas import tpu_sc as plsc

NC, NS, L = 2, 16, 16
B = 16
V = 100001
RNN = 2048
HALF = 50000
WIN = 50048
UNROLL = 8
TOTAL = B * V
NEG = -1e10
NINF = -3e38
TOKBITS = 17

NBLK = WIN // (UNROLL * L)
NBLK2 = (NBLK + UNROLL - 1) // UNROLL
CHUNK_BLKS = (98, 98, 98, 97)

_mesh = plsc.VectorSubcoreMesh(core_axis_name="c", subcore_axis_name="s")


def _rev(x):
    return lax.rev(x, (0,))


def _merge_sorted(av, ai, bv, bi):
    rbv, rbi = _rev(bv), _rev(bi)
    take = rbv > av
    nv = jnp.where(take, rbv, av)
    ni = jnp.where(take, rbi, ai)
    return plsc.sort_key_val(nv, ni)


def _insert16(tv, ti, v, vi):
    sv, si = plsc.sort_key_val(v, vi)
    rsv, rsi = _rev(sv), _rev(si)
    take = rsv > tv
    nv = jnp.where(take, rsv, tv)
    ni = jnp.where(take, rsi, ti)
    tv2, ti2 = plsc.sort_key_val(nv, ni)
    return tv2, ti2, tv2[0]


@functools.partial(
    pl.kernel,
    out_type=(
        jax.ShapeDtypeStruct((NS, NC, L), jnp.float32),
        jax.ShapeDtypeStruct((NS, NC, L), jnp.int32),
    ),
    mesh=_mesh,
    scratch_types=[
        pltpu.VMEM((WIN,), jnp.float32),
        pltpu.VMEM((NBLK2 * UNROLL * L,), jnp.float32),
        pltpu.VMEM((NBLK2 * L,), jnp.float32),
        pltpu.VMEM((L,), jnp.int32),
        pltpu.VMEM((L,), jnp.float32),
        pltpu.VMEM((L,), jnp.int32),
        pltpu.SemaphoreType.DMA,
        pltpu.SemaphoreType.DMA,
        pltpu.SemaphoreType.DMA,
        pltpu.SemaphoreType.DMA,
    ],
    compiler_params=pltpu.CompilerParams(needs_layout_passes=False),
)
def _scan(lp_hbm, forbid_hbm, ovals_hbm, oidx_hbm,
          buf, g1, g2, prev_v, vals_v, idx_v, s0, s1, s2, s3):
    c = lax.axis_index("c")
    s = lax.axis_index("s")
    lane = lax.iota(jnp.int32, L)
    negs = jnp.full((L,), NEG, jnp.float32)

    row_base = s * V
    lo = row_base + c * HALF
    astart = jnp.minimum((lo // 8) * 8, TOTAL - WIN)
    d = lo - astart

    sems = (s0, s1, s2, s3)
    copies = []
    base = 0
    for nb in CHUNK_BLKS:
        n = nb * UNROLL * L
        copies.append(pltpu.async_copy(
            lp_hbm.at[pl.ds(astart + base, n)], buf.at[pl.ds(base, n)],
            sems[len(copies)]))
        base += n

    pltpu.sync_copy(forbid_hbm.at[pl.ds(s * L, L)], prev_v)
    prev_flat = prev_v[...]
    pidx = prev_flat - astart

    def pass_a(i, _):
        bb = i * (UNROLL * L)
        g = buf[pl.ds(bb, L)]
        for k in range(1, UNROLL):
            g = jnp.maximum(g, buf[pl.ds(bb + k * L, L)])
        g1[pl.ds(i * L, L)] = g
        return 0

    blk0 = 0
    base = 0
    for ci, nb in enumerate(CHUNK_BLKS):
        n = nb * UNROLL * L
        copies[ci].wait()
        if ci == 0:
            for j in range(3):
                idxh = j * L + lane
                plsc.store_scatter(buf, [idxh], negs, mask=idxh < d)
        if ci == len(CHUNK_BLKS) - 1:
            for j in range(3):
                idxt = (HALF // L + j) * L + lane
                plsc.store_scatter(buf, [idxt], negs,
                                   mask=idxt >= HALF + d)
        pmask = (pidx >= base) & (pidx < base + n)
        plsc.store_scatter(buf, [jnp.clip(pidx, 0, WIN - 1)], negs,
                           mask=pmask)
        lax.fori_loop(blk0, blk0 + nb, pass_a, 0)
        blk0 += nb
        base += n

    g1[pl.ds(NBLK * L, L)] = negs

    def pass_a2(i, _):
        bb = i * (UNROLL * L)
        g = g1[pl.ds(bb, L)]
        for k in range(1, UNROLL):
            g = jnp.maximum(g, g1[pl.ds(bb + k * L, L)])
        g2[pl.ds(i * L, L)] = g
        return 0

    lax.fori_loop(0, NBLK2, pass_a2, 0)

    def pass_b(i, tvb):
        sv = jnp.sort(g2[pl.ds(i * L, L)])
        nv = jnp.maximum(tvb, _rev(sv))
        return jnp.sort(nv)

    tvb = lax.fori_loop(0, NBLK2, pass_b,
                        jnp.full((L,), NINF, jnp.float32))
    thr = tvb[0]

    def hits(v):
        return plsc.all_reduce_population_count(v >= thr)[0]

    def pass_c(j, carry):
        def descend(c2):
            for a in range(UNROLL):
                gi = j * UNROLL + a

                def down2(c3, gi=gi):
                    for k in range(UNROLL):
                        bb = (gi * UNROLL + k) * L

                        def ins(c4, bb=bb):
                            fvec = (astart + bb) + lane
                            v = buf[pl.ds(bb, L)]
                            return _insert16(c4[0], c4[1], v,
                                             fvec - row_base)[:2]

                        c3 = lax.cond(hits(buf[pl.ds(bb, L)]) > 0,
                                      ins, lambda c4: c4, c3)
                    return c3

                c2 = lax.cond((gi < NBLK) & (hits(g1[pl.ds(gi * L, L)]) > 0),
                              down2, lambda c3: c3, c2)
            return c2

        return lax.cond(hits(g2[pl.ds(j * L, L)]) > 0,
                        descend, lambda c2: c2, carry)

    init = (jnp.full((L,), NINF, jnp.float32), jnp.zeros((L,), jnp.int32))
    tv, ti = lax.fori_loop(0, NBLK2, pass_c, init)

    vals_v[...] = tv
    idx_v[...] = ti
    pltpu.sync_copy(vals_v, ovals_hbm.at[s, c])
    pltpu.sync_copy(idx_v, oidx_hbm.at[s, c])


@functools.partial(
    pl.kernel,
    out_type=(
        jax.ShapeDtypeStruct((B,), jnp.int32),
        jax.ShapeDtypeStruct((B,), jnp.float32),
        jax.ShapeDtypeStruct((B,), jnp.float32),
        jax.ShapeDtypeStruct((2 * B, RNN), jnp.float32),
        jax.ShapeDtypeStruct((2 * B, RNN), jnp.float32),
    ),
    mesh=_mesh,
    scratch_types=[
        pltpu.VMEM((NC, L), jnp.float32),
        pltpu.VMEM((NC, L), jnp.int32),
        pltpu.VMEM((L,), jnp.float32),
        pltpu.VMEM((L,), jnp.float32),
        pltpu.VMEM((B * L,), jnp.float32),
        pltpu.VMEM((L,), jnp.float32),
        pltpu.VMEM((L,), jnp.int32),
        pltpu.VMEM((L,), jnp.int32),
        pltpu.VMEM((L,), jnp.float32),
        pltpu.VMEM((L,), jnp.float32),
        pltpu.VMEM((4,), jnp.int32),
        pltpu.VMEM((4, RNN), jnp.float32),
        pltpu.VMEM((17 * L,), jnp.float32),
        pltpu.VMEM((17 * L,), jnp.int32),
        pltpu.VMEM((8 * NS,), jnp.int32),
        pltpu.VMEM_SHARED((17 * L,), jnp.float32),
        pltpu.VMEM_SHARED((17 * L,), jnp.int32),
        pltpu.VMEM_SHARED((8 * NS,), jnp.int32),
        pltpu.SemaphoreType.DMA,
    ],
    compiler_params=pltpu.CompilerParams(needs_layout_passes=False),
)
def _merge(vals_hbm, idx_hbm, bsum_hbm, bsumb_hbm, eos_hbm, prev_hbm,
           sh_hbm, sc_hbm,
           tok_out, p_out, r_out, nh_out, nc_out,
           hv_v, hi_v, bs_v, bsum_v, bsumb_v, eos_v, prev_v, tok_v, p_v, r_v,
           q4_v, rows_v, mv_v, mi_v, q2_v, shv, shi, qsh, sem):
    c = lax.axis_index("c")
    s = lax.axis_index("s")
    lane = lax.iota(jnp.int32, L)

    @pl.when(c == 0)
    def _core0():
        pltpu.sync_copy(vals_hbm.at[s], hv_v)
        pltpu.sync_copy(idx_hbm.at[s], hi_v)
        pltpu.sync_copy(bsumb_hbm.at[pl.ds(s * L, L)], bs_v)
        mv, mi = _merge_sorted(hv_v[0], hi_v[0], hv_v[1], hi_v[1])
        p_v[...] = mv + bs_v[...]
        tok_v[...] = mi | lax.shift_left(s, TOKBITS)
        pltpu.sync_copy(p_v, shv.at[pl.ds(s * L, L)])
        pltpu.sync_copy(tok_v, shi.at[pl.ds(s * L, L)])

        @pl.when(s == 0)
        def _eos():
            pltpu.sync_copy(eos_hbm, eos_v)
            pltpu.sync_copy(prev_hbm, prev_v)
            pltpu.sync_copy(bsum_hbm, bsum_v)
            pltpu.sync_copy(bsumb_hbm, bsumb_v)
            ev = (jnp.where(prev_v[...] == V - 1, NEG, eos_v[...])
                  - 1000.0 + bsum_v[...])
            epk = jnp.left_shift(lane, TOKBITS) | (V - 1)
            sv, si = plsc.sort_key_val(ev, epk)
            p_v[...] = sv
            tok_v[...] = si
            pltpu.sync_copy(p_v, shv.at[pl.ds(B * L, L)])
            pltpu.sync_copy(tok_v, shi.at[pl.ds(B * L, L)])

        plsc.subcore_barrier()

        @pl.when(s < 4)
        def _fold4():
            pltpu.sync_copy(shv.at[pl.ds(4 * s * L, 4 * L)],
                            mv_v.at[pl.ds(0, 4 * L)])
            pltpu.sync_copy(shi.at[pl.ds(4 * s * L, 4 * L)],
                            mi_v.at[pl.ds(0, 4 * L)])
            av = mv_v[pl.ds(0, L)]
            ai = mi_v[pl.ds(0, L)]
            for k in range(1, 4):
                av, ai = _merge_sorted(av, ai, mv_v[pl.ds(k * L, L)],
                                       mi_v[pl.ds(k * L, L)])
            p_v[...] = av
            tok_v[...] = ai
            pltpu.sync_copy(p_v, shv.at[pl.ds(4 * s * L, L)])
            pltpu.sync_copy(tok_v, shi.at[pl.ds(4 * s * L, L)])

        plsc.subcore_barrier()

        @pl.when(s == 0)
        def _lead():
            pltpu.sync_copy(shv, mv_v)
            pltpu.sync_copy(shi, mi_v)
            m1v, m1i = _merge_sorted(mv_v[pl.ds(0, L)], mi_v[pl.ds(0, L)],
                                     mv_v[pl.ds(4 * L, L)],
                                     mi_v[pl.ds(4 * L, L)])
            m2v, m2i = _merge_sorted(mv_v[pl.ds(8 * L, L)],
                                     mi_v[pl.ds(8 * L, L)],
                                     mv_v[pl.ds(12 * L, L)],
                                     mi_v[pl.ds(12 * L, L)])
            gv, gi = _merge_sorted(m1v, m1i, m2v, m2i)
            gv, gi = _merge_sorted(gv, gi, mv_v[pl.ds(B * L, L)],
                                   mi_v[pl.ds(B * L, L)])
            top_p = _rev(gv)
            pk = _rev(gi)
            token = pk & ((1 << TOKBITS) - 1)
            q = lax.shift_right_logical(pk, TOKBITS)
            bq = jnp.zeros((L,), jnp.float32)
            for b in range(B):
                bq = jnp.where(q == b, bsumb_v[pl.ds(b * L, L)], bq)
            r = top_p - bq

            tok_v[...] = token
            p_v[...] = top_p
            r_v[...] = r
            dest = 8 * (lane // 4) + (lane % 4)
            plsc.store_scatter(q2_v, [dest], q)
            plsc.store_scatter(q2_v, [dest + 32], q + B)
            plsc.store_scatter(q2_v, [dest + 64], q)
            plsc.store_scatter(q2_v, [dest + 96], q + B)
            pltpu.sync_copy(q2_v, qsh)
            pltpu.sync_copy(tok_v, tok_out)
            pltpu.sync_copy(p_v, p_out)
            pltpu.sync_copy(r_v, r_out)

        plsc.subcore_barrier()

        for k in range(NS):
            @pl.when(s == k)
            def _gather(k=k):
                pltpu.sync_copy(qsh.at[pl.ds(8 * k, 4)], q4_v)
                src = sh_hbm if k < 8 else sc_hbm
                dst = nh_out if k < 8 else nc_out
                dbase = 4 * k if k < 8 else 4 * k - 32
                pltpu.async_copy(src.at[q4_v], rows_v, sem).wait()
                pltpu.sync_copy(rows_v, dst.at[pl.ds(dbase, 4)])


def kernel(logprobs, beam_logprobs_sum, state_h, state_c, prev_tokens):
    lp = logprobs.astype(jnp.float32)
    prev = prev_tokens.astype(jnp.int32)
    bsum = beam_logprobs_sum.astype(jnp.float32)
    eos = lp[:, V - 1]
    forbid = jnp.broadcast_to(
        (prev + jnp.arange(B, dtype=jnp.int32) * V)[:, None], (B, L)
    ).reshape(-1)
    bsumb = jnp.broadcast_to(bsum[:, None], (B, L)).reshape(-1)
    vals, idx = _scan(lp.reshape(-1), forbid)
    token, top_p, r, nh, nc = _merge(
        vals, idx, bsum, bsumb, eos, prev,
        state_h.reshape(2 * B, RNN), state_c.reshape(2 * B, RNN))
    return (token, top_p, r,
            nh.reshape(2, B, RNN), nc.reshape(2, B, RNN))
